# baseline (speedup 1.0000x reference)
import functools
import math

import jax
import jax.numpy as jnp
from jax import lax
from jax.experimental import pallas as pl
from jax.experimental.pallas import tpu as pltpu
from jax.experimental.pallas import tpu_sc as plsc

B = 16384
F = 26
V = 100000
D = 16
FD = F * D
ROWS = B * F
BN_SCALE = float(1.0 / math.sqrt(1.0 + 1e-5))

_NC = 2
_NS = 16
_NW = _NC * _NS
_PER_W = ROWS // _NW
_SPW = 128
_KS = 8
_CHUNK = _SPW * _KS
_NCHUNK = _PER_W // _CHUNK


def _sc_gather_body(tbl2_hbm, tbl1_hbm, idx_hbm, e2_hbm, e1_hbm,
                    idx_v, rows_v, e1_v, sem2, sem1):
    wid = lax.axis_index("s") * _NC + lax.axis_index("c")
    row0 = wid * (_PER_W // _SPW)

    def chunk_body(c, carry):
        base128 = row0 + c * _KS
        off = base128 * _SPW
        pltpu.sync_copy(idx_hbm.at[pl.ds(base128, _KS)], idx_v)
        cps = []
        for j in range(_KS):
            cps.append(pltpu.async_copy(tbl2_hbm.at[idx_v.at[j]], rows_v.at[j], sem2))
            cps.append(pltpu.async_copy(tbl1_hbm.at[idx_v.at[j]], e1_v.at[j], sem1))
        for cp in cps:
            cp.wait()
        for j in range(_KS):
            pltpu.sync_copy(rows_v.at[j], e2_hbm.at[pl.ds(off + j * _SPW, _SPW)])
            pltpu.sync_copy(e1_v.at[j], e1_hbm.at[pl.ds(off + j * _SPW, _SPW)])
        return carry

    lax.fori_loop(0, _NCHUNK, chunk_body, 0)


def _make_sc_gather():
    return functools.partial(
        pl.kernel,
        mesh=plsc.VectorSubcoreMesh(core_axis_name="c", subcore_axis_name="s"),
        compiler_params=pltpu.CompilerParams(use_tc_tiling_on_sc=False),
        out_type=[
            jax.ShapeDtypeStruct((ROWS, D), jnp.float32),
            jax.ShapeDtypeStruct((ROWS,), jnp.float32),
        ],
        scratch_types=[
            pltpu.VMEM((_KS, _SPW), jnp.int32),
            pltpu.VMEM((_KS, _SPW, D), jnp.float32),
            pltpu.VMEM((_KS, _SPW), jnp.float32),
            pltpu.SemaphoreType.DMA,
            pltpu.SemaphoreType.DMA,
        ],
    )(_sc_gather_body)


def _tc_body(e2_ref, xv_ref, e1_ref, r_ref, s_ref, st_ref,
             w1_ref, b1_ref, w2_ref, b2_ref, w3_ref, b3_ref,
             wdt_ref, bd_ref, out_ref):
    hi = jax.lax.Precision.HIGHEST
    e2 = e2_ref[...]
    xv_raw = xv_ref[...]
    e1 = e1_ref[...]
    r_m = r_ref[...]
    s_m = s_ref[...]
    st_m = st_ref[...]

    u = e2 * jnp.dot(xv_raw, r_m, precision=hi)
    ss = jnp.dot(u * u, s_m, precision=hi)
    inv = 1.0 / jnp.maximum(jnp.sqrt(ss), 1e-12)
    xvn = u * jnp.dot(inv, st_m, precision=hi)

    t = jnp.dot(xvn, s_m, precision=hi)
    s2 = jnp.dot(xvn * xvn, s_m, precision=hi)
    f2s = 0.5 * jnp.sum(t * t - s2, axis=1, keepdims=True)
    f1s = jnp.sum(e1 * xv_raw, axis=1, keepdims=True)

    h = xvn
    for w_ref, b_ref in ((w1_ref, b1_ref), (w2_ref, b2_ref), (w3_ref, b3_ref)):
        h = jnp.maximum((jnp.dot(h, w_ref[...], precision=hi) + b_ref[...]) * BN_SCALE, 0.0)
    dsum = jnp.sum(h, axis=1, keepdims=True)

    wdt = wdt_ref[...]
    out_ref[...] = (f1s * wdt[0:1, :] + f2s * wdt[1:2, :] + dsum * wdt[2:3, :]
                    + bd_ref[...])


_BLK = 1024
_GRID = B // _BLK


def _tc_call(e2, xv, e1, r_m, s_m, st_m, w1t, b1r, w2t, b2r, w3t, b3r, wdt, bdr):
    def blk(shape):
        return pl.BlockSpec(shape, lambda i: (0, 0))

    return pl.pallas_call(
        _tc_body,
        grid=(_GRID,),
        in_specs=[
            pl.BlockSpec((_BLK, FD), lambda i: (i, 0)),
            pl.BlockSpec((_BLK, F), lambda i: (i, 0)),
            pl.BlockSpec((_BLK, F), lambda i: (i, 0)),
            blk((F, FD)),
            blk((FD, D)),
            blk((D, FD)),
            blk((FD, 200)),
            blk((1, 200)),
            blk((200, 200)),
            blk((1, 200)),
            blk((200, 200)),
            blk((1, 200)),
            blk((3, 2)),
            blk((1, 2)),
        ],
        out_specs=pl.BlockSpec((_BLK, 2), lambda i: (i, 0)),
        out_shape=jax.ShapeDtypeStruct((B, 2), jnp.float32),
    )(e2, xv, e1, r_m, s_m, st_m, w1t, b1r, w2t, b2r, w3t, b3r, wdt, bdr)


def kernel(Xi, Xv, tbl1, tbl2, W1, b1, W2, b2, W3, b3, Wd, bd):
    idx = Xi[:, :, 0].astype(jnp.int32)
    flat_idx = (jnp.arange(F, dtype=jnp.int32)[None, :] * V + idx)
    flat_idx = flat_idx.reshape(ROWS // _SPW, _SPW)
    tbl2f = tbl2.reshape(F * V, D)
    tbl1f = tbl1.reshape(F * V)

    e2_flat, e1_flat = _make_sc_gather()(tbl2f, tbl1f, flat_idx)
    e2 = e2_flat.reshape(B, FD)
    e1 = e1_flat.reshape(B, F)

    lanes = jnp.arange(FD, dtype=jnp.int32)
    r_m = (lanes[None, :] // D == jnp.arange(F, dtype=jnp.int32)[:, None]).astype(jnp.float32)
    s_m = (lanes[:, None] % D == jnp.arange(D, dtype=jnp.int32)[None, :]).astype(jnp.float32)
    st_m = s_m.T

    return _tc_call(
        e2, Xv, e1, r_m, s_m, st_m,
        W1.T, b1.reshape(1, 200), W2.T, b2.reshape(1, 200), W3.T, b3.reshape(1, 200),
        Wd.T, bd.reshape(1, 2),
    )


# speedup vs baseline: 4.4537x; 4.4537x over previous
import functools
import math

import jax
import jax.numpy as jnp
from jax import lax
from jax.experimental import pallas as pl
from jax.experimental.pallas import tpu as pltpu
from jax.experimental.pallas import tpu_sc as plsc

B = 16384
F = 26
V = 100000
D = 16
FD = F * D
BN_SCALE = float(1.0 / math.sqrt(1.0 + 1e-5))

_NC = 2
_NS = 16
_NW = _NC * _NS
_PPW = FD // _NW
_OCH = 8192
_L = 16
_UNROLL = 8


def _sc_gather_body(t2_hbm, t1_hbm, idx_hbm, e2t_hbm, e1t_hbm,
                    plane_v, idx_v, out_v, sem):
    wid = lax.axis_index("s") * _NC + lax.axis_index("c")

    def gather_plane(row_ref, out_row_ref):
        pltpu.async_copy(row_ref, plane_v, sem).wait()
        for c in range(B // _OCH):
            def chunk_iter(i, carry):
                for u in range(_UNROLL):
                    g = c * _OCH + (i * _UNROLL + u) * _L
                    iv = idx_v[g // 128, pl.ds(g % 128, _L)]
                    vals = plsc.load_gather(plane_v, [iv])
                    out_v[pl.ds((i * _UNROLL + u) * _L, _L)] = vals
                return carry
            lax.fori_loop(0, _OCH // _L // _UNROLL, chunk_iter, 0)
            pltpu.sync_copy(out_v, out_row_ref.at[pl.ds(c * _OCH, _OCH)])

    def plane_body(p, carry):
        r = wid * _PPW + p
        f = r // D
        pltpu.sync_copy(idx_hbm.at[f], idx_v)
        gather_plane(t2_hbm.at[r], e2t_hbm.at[r])
        return carry

    lax.fori_loop(0, _PPW, plane_body, 0)

    @pl.when(wid < F)
    def _():
        pltpu.sync_copy(idx_hbm.at[wid], idx_v)
        gather_plane(t1_hbm.at[wid], e1t_hbm.at[wid])


def _make_sc_gather():
    return functools.partial(
        pl.kernel,
        mesh=plsc.VectorSubcoreMesh(core_axis_name="c", subcore_axis_name="s"),
        compiler_params=pltpu.CompilerParams(needs_layout_passes=False),
        out_type=[
            jax.ShapeDtypeStruct((FD, B), jnp.float32),
            jax.ShapeDtypeStruct((F, B), jnp.float32),
        ],
        scratch_types=[
            pltpu.VMEM((V,), jnp.float32),
            pltpu.VMEM((B // 128, 128), jnp.int32),
            pltpu.VMEM((_OCH,), jnp.float32),
            pltpu.SemaphoreType.DMA,
        ],
    )(_sc_gather_body)


def _tc_body(e2t_ref, e1t_ref, xvt_ref, rf_ref, s2_ref, e16_ref,
             w1_ref, b1_ref, w2_ref, b2_ref, w3_ref, b3_ref,
             wd_ref, bd_ref, out_ref):
    hi = jax.lax.Precision.HIGHEST
    e2t = e2t_ref[...]
    e1t = e1t_ref[...]
    xvt = xvt_ref[...]
    rf = rf_ref[...]
    s2 = s2_ref[...]
    e16 = e16_ref[...]

    u = e2t * jnp.dot(rf, xvt, precision=hi)
    ss = jnp.dot(s2, u * u, precision=hi)
    inv = 1.0 / jnp.maximum(jnp.sqrt(ss), 1e-12)
    xvn = u * jnp.dot(e16, inv, precision=hi)

    t = jnp.dot(s2, xvn, precision=hi)
    s2sum = jnp.dot(s2, xvn * xvn, precision=hi)
    f2s = 0.5 * jnp.sum(t * t - s2sum, axis=0, keepdims=True)
    f1s = jnp.sum(e1t * xvt, axis=0, keepdims=True)

    h = xvn
    for w_ref, b_ref in ((w1_ref, b1_ref), (w2_ref, b2_ref), (w3_ref, b3_ref)):
        h = jnp.maximum((jnp.dot(w_ref[...], h, precision=hi) + b_ref[...]) * BN_SCALE, 0.0)
    dsum = jnp.sum(h, axis=0, keepdims=True)

    stacked = jnp.concatenate([f1s, f2s, dsum], axis=0)
    out_ref[...] = jnp.dot(wd_ref[...], stacked, precision=hi) + bd_ref[...]


_BLK = 1024
_GRID = B // _BLK


def _tc_call(e2t, e1t, xvt, rf, s2, e16, w1, b1c, w2, b2c, w3, b3c, wd, bdc):
    def full(shape):
        return pl.BlockSpec(shape, lambda i: (0, 0))

    return pl.pallas_call(
        _tc_body,
        grid=(_GRID,),
        in_specs=[
            pl.BlockSpec((FD, _BLK), lambda i: (0, i)),
            pl.BlockSpec((F, _BLK), lambda i: (0, i)),
            pl.BlockSpec((F, _BLK), lambda i: (0, i)),
            full((FD, F)),
            full((D, FD)),
            full((FD, D)),
            full((200, FD)),
            full((200, 1)),
            full((200, 200)),
            full((200, 1)),
            full((200, 200)),
            full((200, 1)),
            full((2, 3)),
            full((2, 1)),
        ],
        out_specs=pl.BlockSpec((2, _BLK), lambda i: (0, i)),
        out_shape=jax.ShapeDtypeStruct((2, B), jnp.float32),
    )(e2t, e1t, xvt, rf, s2, e16, w1, b1c, w2, b2c, w3, b3c, wd, bdc)


def kernel(Xi, Xv, tbl1, tbl2, W1, b1, W2, b2, W3, b3, Wd, bd):
    idx = Xi[:, :, 0].astype(jnp.int32)
    idx_t = idx.T.reshape(F, B // 128, 128)
    t2 = jnp.transpose(tbl2, (0, 2, 1)).reshape(FD, V)
    t1 = tbl1[:, :, 0]

    e2t, e1t = _make_sc_gather()(t2, t1, idx_t)

    rows = jnp.arange(FD, dtype=jnp.int32)
    rf = (rows[:, None] // D == jnp.arange(F, dtype=jnp.int32)[None, :]).astype(jnp.float32)
    s2 = (rows[None, :] % D == jnp.arange(D, dtype=jnp.int32)[:, None]).astype(jnp.float32)
    e16 = s2.T

    out_t = _tc_call(
        e2t, e1t, Xv.T, rf, s2, e16,
        W1, b1.reshape(200, 1), W2, b2.reshape(200, 1), W3, b3.reshape(200, 1),
        Wd, bd.reshape(2, 1),
    )
    return out_t.T


# speedup vs baseline: 6.5357x; 1.4675x over previous
import functools
import math

import jax
import jax.numpy as jnp
from jax import lax
from jax.experimental import pallas as pl
from jax.experimental.pallas import tpu as pltpu
from jax.experimental.pallas import tpu_sc as plsc

B = 16384
F = 26
V = 100000
D = 16
FD = F * D
BN_SCALE = float(1.0 / math.sqrt(1.0 + 1e-5))

_NC = 2
_NS = 16
_NW = _NC * _NS
_PPW = FD // _NW
_OCH = 8192
_L = 16
_UNROLL = 8


def _sc_gather_body(t2_hbm, t1_hbm, idx_hbm, e2t_hbm, e1t_hbm,
                    plane_v, idx_v, out_v, sem):
    wid = lax.axis_index("s") * _NC + lax.axis_index("c")

    def gather_plane(row_ref, out_row_ref):
        pltpu.async_copy(row_ref, plane_v, sem).wait()
        for c in range(B // _OCH):
            def chunk_iter(i, carry):
                for u in range(_UNROLL):
                    g = c * _OCH + (i * _UNROLL + u) * _L
                    iv = idx_v[g // 128, pl.ds(g % 128, _L)]
                    vals = plsc.load_gather(plane_v, [iv])
                    out_v[pl.ds((i * _UNROLL + u) * _L, _L)] = vals
                return carry
            lax.fori_loop(0, _OCH // _L // _UNROLL, chunk_iter, 0)
            pltpu.sync_copy(out_v, out_row_ref.at[pl.ds(c * _OCH, _OCH)])

    def plane_body(p, carry):
        r = wid * _PPW + p
        f = r // D
        pltpu.sync_copy(idx_hbm.at[f], idx_v)
        gather_plane(t2_hbm.at[r], e2t_hbm.at[r])
        return carry

    lax.fori_loop(0, _PPW, plane_body, 0)

    @pl.when(wid < F)
    def _():
        pltpu.sync_copy(idx_hbm.at[wid], idx_v)
        gather_plane(t1_hbm.at[wid], e1t_hbm.at[wid])


def _make_sc_gather():
    return functools.partial(
        pl.kernel,
        mesh=plsc.VectorSubcoreMesh(core_axis_name="c", subcore_axis_name="s"),
        compiler_params=pltpu.CompilerParams(needs_layout_passes=False),
        out_type=[
            jax.ShapeDtypeStruct((FD, B), jnp.float32),
            jax.ShapeDtypeStruct((F, B), jnp.float32),
        ],
        scratch_types=[
            pltpu.VMEM((V,), jnp.float32),
            pltpu.VMEM((B // 128, 128), jnp.int32),
            pltpu.VMEM((_OCH,), jnp.float32),
            pltpu.SemaphoreType.DMA,
        ],
    )(_sc_gather_body)


def _tc_body(e2t_ref, e1t_ref, xvt_ref,
             w1_ref, b1_ref, w2_ref, b2_ref, w3_ref, b3_ref,
             wd_ref, bd_ref, out_ref):
    hi = jax.lax.Precision.HIGHEST
    e2t = e2t_ref[...]
    e1t = e1t_ref[...]
    xvt = xvt_ref[...]

    e3 = e2t.reshape(F, D, _BLK)
    u = e3 * xvt[:, None, :]
    ss = jnp.sum(u * u, axis=0)
    inv = 1.0 / jnp.maximum(jnp.sqrt(ss), 1e-12)
    xvn3 = u * inv[None, :, :]

    t = jnp.sum(xvn3, axis=0)
    s2sum = jnp.sum(xvn3 * xvn3, axis=0)
    f2s = jnp.sum(0.5 * (t * t - s2sum), axis=0, keepdims=True)
    f1s = jnp.sum(e1t * xvt, axis=0, keepdims=True)

    h = xvn3.reshape(FD, _BLK)
    for w_ref, b_ref in ((w1_ref, b1_ref), (w2_ref, b2_ref), (w3_ref, b3_ref)):
        h = jnp.maximum((jnp.dot(w_ref[...], h) + b_ref[...]) * BN_SCALE, 0.0)
    dsum = jnp.sum(h, axis=0, keepdims=True)

    stacked = jnp.concatenate([f1s, f2s, dsum], axis=0)
    out_ref[...] = jnp.dot(wd_ref[...], stacked, precision=hi) + bd_ref[...]


_BLK = 1024
_GRID = B // _BLK


def _tc_call(e2t, e1t, xvt, w1, b1c, w2, b2c, w3, b3c, wd, bdc):
    def full(shape):
        return pl.BlockSpec(shape, lambda i: (0, 0))

    return pl.pallas_call(
        _tc_body,
        grid=(_GRID,),
        in_specs=[
            pl.BlockSpec((FD, _BLK), lambda i: (0, i)),
            pl.BlockSpec((F, _BLK), lambda i: (0, i)),
            pl.BlockSpec((F, _BLK), lambda i: (0, i)),
            full((200, FD)),
            full((200, 1)),
            full((200, 200)),
            full((200, 1)),
            full((200, 200)),
            full((200, 1)),
            full((2, 3)),
            full((2, 1)),
        ],
        out_specs=pl.BlockSpec((2, _BLK), lambda i: (0, i)),
        out_shape=jax.ShapeDtypeStruct((2, B), jnp.float32),
    )(e2t, e1t, xvt, w1, b1c, w2, b2c, w3, b3c, wd, bdc)


def kernel(Xi, Xv, tbl1, tbl2, W1, b1, W2, b2, W3, b3, Wd, bd):
    idx = Xi[:, :, 0].astype(jnp.int32)
    idx_t = idx.T.reshape(F, B // 128, 128)
    t2 = jnp.transpose(tbl2, (0, 2, 1)).reshape(FD, V)
    t1 = tbl1[:, :, 0]

    e2t, e1t = _make_sc_gather()(t2, t1, idx_t)

    out_t = _tc_call(
        e2t, e1t, Xv.T,
        W1, b1.reshape(200, 1), W2, b2.reshape(200, 1), W3, b3.reshape(200, 1),
        Wd, bd.reshape(2, 1),
    )
    return out_t.T
